# baseline (speedup 1.0000x reference)
import functools

import jax
import jax.numpy as jnp
from jax import lax
from jax.experimental import pallas as pl
from jax.experimental.pallas import tpu as pltpu
from jax.experimental.pallas import tpu_sc as plsc

_G = 128


@functools.partial(jax.jit, static_argnums=(2, 3, 4))
def _gather_sc(xT, table, J, N, D):
    info = plsc.get_sparse_core_info()
    NC, NS = info.num_cores, info.num_subcores
    NW = NC * NS
    mesh = plsc.VectorSubcoreMesh(core_axis_name="c", subcore_axis_name="s")
    n_rows = J * D * NW

    @functools.partial(
        pl.kernel,
        mesh=mesh,
        out_type=jax.ShapeDtypeStruct((n_rows, _G), jnp.float32),
        scratch_types=[
            pltpu.VMEM((2, _G), jnp.int32),
            pltpu.VMEM((_G, D), jnp.float32),
            pltpu.VMEM((_G, D), jnp.float32),
            pltpu.VMEM((D, _G), jnp.float32),
            pltpu.VMEM((D, _G), jnp.float32),
            pltpu.VMEM((D,), jnp.int32),
            pltpu.VMEM((D,), jnp.int32),
            pltpu.SemaphoreType.DMA,
            pltpu.SemaphoreType.DMA,
            pltpu.SemaphoreType.DMA,
        ],
        compiler_params=pltpu.CompilerParams(
            use_tc_tiling_on_sc=False,
            needs_layout_passes=False,
            skip_device_barrier=True,
        ),
    )
    def _k(xT_hbm, table_hbm, out_hbm, idx_v, r0, r1, t0, t1, iw0, iw1,
           sg0, sg1, sw):
        wid = lax.axis_index("s") * NC + lax.axis_index("c")
        off = wid * _G
        iotav = lax.iota(jnp.int32, 16)
        perms = [(iotav + s) % 16 for s in range(16)]

        def transpose_block(r, t):
            def dbody(db, carry):
                d0 = 16 * db
                for g in range(_G // 16):
                    rows = 16 * g + iotav
                    srcs = [
                        plsc.load_gather(r, [rows, d0 + perms[s]])
                        for s in range(16)
                    ]
                    for s in range(16):
                        plsc.store_scatter(t, [d0 + perms[s], rows], srcs[s])
                return carry

            lax.fori_loop(0, D // 16, dbody, 0)

        def set_scatter_rows(iw, j):
            base = j * (D * NW) + wid
            for q in range(D // 16):
                iw[pl.ds(16 * q, 16)] = (16 * q + iotav) * NW + base

        def body(p, carry):
            j0 = 2 * p
            pltpu.sync_copy(xT_hbm.at[pl.ds(j0, 2), pl.ds(off, _G)], idx_v)
            g0 = pltpu.async_copy(table_hbm.at[idx_v.at[0]], r0, sg0)
            g1 = pltpu.async_copy(table_hbm.at[idx_v.at[1]], r1, sg1)
            set_scatter_rows(iw0, j0)
            set_scatter_rows(iw1, j0 + 1)
            g0.wait()
            transpose_block(r0, t0)
            w0 = pltpu.async_copy(t0, out_hbm.at[iw0], sw)
            g1.wait()
            transpose_block(r1, t1)
            w1 = pltpu.async_copy(t1, out_hbm.at[iw1], sw)
            w0.wait()
            w1.wait()
            return carry

        lax.fori_loop(0, J // 2, body, 0)

    return _k(xT, table)


def kernel(x, table):
    B, S = x.shape
    D = table.shape[1]
    xT = x.T.astype(jnp.int32)
    y = _gather_sc(xT, table, S, B, D)
    y = y.reshape(S, D, B)
    return jnp.transpose(y, (2, 0, 1))


# speedup vs baseline: 1.2245x; 1.2245x over previous
import functools

import jax
import jax.numpy as jnp
from jax import lax
from jax.experimental import pallas as pl
from jax.experimental.pallas import tpu as pltpu
from jax.experimental.pallas import tpu_sc as plsc

_G = 128


@functools.partial(jax.jit, static_argnums=(2, 3, 4))
def _gather_sc(xT, table, J, N, D):
    info = plsc.get_sparse_core_info()
    NC, NS = info.num_cores, info.num_subcores
    NW = NC * NS
    mesh = plsc.VectorSubcoreMesh(core_axis_name="c", subcore_axis_name="s")
    n_rows = J * D * NW

    @functools.partial(
        pl.kernel,
        mesh=mesh,
        out_type=jax.ShapeDtypeStruct((n_rows, _G), jnp.float32),
        scratch_types=[
            pltpu.VMEM((2, _G), jnp.int32),
            pltpu.VMEM((_G, D), jnp.float32),
            pltpu.VMEM((_G, D), jnp.float32),
            pltpu.VMEM((D, _G), jnp.float32),
            pltpu.VMEM((D, _G), jnp.float32),
            pltpu.VMEM((D,), jnp.int32),
            pltpu.VMEM((D,), jnp.int32),
            pltpu.SemaphoreType.DMA,
            pltpu.SemaphoreType.DMA,
            pltpu.SemaphoreType.DMA,
        ],
        compiler_params=pltpu.CompilerParams(
            use_tc_tiling_on_sc=False,
            needs_layout_passes=False,
            skip_device_barrier=True,
        ),
    )
    def _k(xT_hbm, table_hbm, out_hbm, idx_v, r0, r1, t0, t1, iw0, iw1,
           sg0, sg1, sw):
        wid = lax.axis_index("s") * NC + lax.axis_index("c")
        off = wid * _G
        iotav = lax.iota(jnp.int32, 16)
        perms = [(iotav + s) % 16 for s in range(16)]

        def transpose_block(r, t):
            def dbody(db, carry):
                d0 = 16 * db
                for g in range(_G // 16):
                    rows = 16 * g + iotav
                    srcs = [
                        plsc.load_gather(r, [rows, d0 + perms[s]])
                        for s in range(16)
                    ]
                    for s in range(16):
                        plsc.store_scatter(t, [d0 + perms[s], rows], srcs[s])
                return carry

            lax.fori_loop(0, D // 16, dbody, 0)

        lanepat = (iotav >> 3) * 256 + (iotav & 7)

        def set_scatter_rows(iw, j):
            base = j * (D * NW) + 8 * wid
            for q in range(D // 16):
                iw[pl.ds(16 * q, 16)] = lanepat + (base + 512 * q)

        def body(p, carry):
            j0 = 2 * p
            pltpu.sync_copy(xT_hbm.at[pl.ds(j0, 2), pl.ds(off, _G)], idx_v)
            g0 = pltpu.async_copy(table_hbm.at[idx_v.at[0]], r0, sg0)
            g1 = pltpu.async_copy(table_hbm.at[idx_v.at[1]], r1, sg1)
            set_scatter_rows(iw0, j0)
            set_scatter_rows(iw1, j0 + 1)
            g0.wait()
            transpose_block(r0, t0)
            w0 = pltpu.async_copy(t0, out_hbm.at[iw0], sw)
            g1.wait()
            transpose_block(r1, t1)
            w1 = pltpu.async_copy(t1, out_hbm.at[iw1], sw)
            w0.wait()
            w1.wait()
            return carry

        lax.fori_loop(0, J // 2, body, 0)

    return _k(xT, table)


def kernel(x, table):
    B, S = x.shape
    D = table.shape[1]
    xT = x.T.astype(jnp.int32)
    y = _gather_sc(xT, table, S, B, D)
    y5 = y.reshape(S, D // 8, B // 128, 8, 128)
    out = jnp.transpose(y5, (2, 4, 0, 1, 3))
    return out.reshape(B, S, D)
